# Initial kernel scaffold; baseline (speedup 1.0000x reference)
import jax
import jax.numpy as jnp
from jax.experimental import pallas as pl


def kernel(x, edge_index, W0, b0, W1, b1, W2, b2):
    raise NotImplementedError("write your pallas kernel here")


# speedup vs baseline: 6.3441x; 6.3441x over previous
import functools

import jax
import jax.numpy as jnp
from jax import lax
from jax.experimental import pallas as pl
from jax.experimental.pallas import tpu as pltpu
from jax.experimental.pallas import tpu_sc as plsc

N = 10000
NPAD = 10112
ROWS_T = NPAD // 16
D = 256
DH = 128
E = 160000
EB = 128
EPAD = 163840
ER = EPAD // EB
RPS = ER // 16
RPS_DEG = ER // 32
DEGW = 16
NB = 25
BR = N // NB

_mesh = plsc.VectorSubcoreMesh(core_axis_name="c", subcore_axis_name="s")


@functools.partial(
    pl.kernel,
    out_type=jax.ShapeDtypeStruct((2, NPAD, DEGW), jnp.float32),
    mesh=_mesh,
    scratch_types=[
        pltpu.VMEM_SHARED((NPAD, DEGW), jnp.float32),
        pltpu.VMEM((RPS_DEG, EB), jnp.int32),
        pltpu.VMEM((EB, DEGW), jnp.float32),
        pltpu.VMEM((ROWS_T // 4, DEGW), jnp.float32),
    ],
)
def _deg_kernel(dst_hbm, deg_hbm, acc, dst_v, ones_v, z_v):
    c = lax.axis_index("c")
    s = lax.axis_index("s")

    def _fill_z(r, carry):
        z_v[r, :] = jnp.zeros((DEGW,), jnp.float32)
        return carry

    def _fill_1(r, carry):
        ones_v[r, :] = jnp.ones((DEGW,), jnp.float32)
        return carry

    lax.fori_loop(0, ROWS_T // 4, _fill_z, 0)
    lax.fori_loop(0, EB, _fill_1, 0)

    for k in range(4):
        pltpu.sync_copy(z_v, acc.at[pl.ds(s * ROWS_T + k * (ROWS_T // 4), ROWS_T // 4)])
    pltpu.sync_copy(dst_hbm.at[pl.ds(c * (ER // 2) + s * RPS_DEG, RPS_DEG)], dst_v)
    plsc.subcore_barrier()

    def _chunk(j, carry):
        pltpu.sync_copy(ones_v, acc.at[dst_v.at[j]], add=True)
        return carry

    lax.fori_loop(0, RPS_DEG, _chunk, 0)
    plsc.subcore_barrier()
    pltpu.sync_copy(acc.at[pl.ds(s * ROWS_T, ROWS_T)],
                    deg_hbm.at[c, pl.ds(s * ROWS_T, ROWS_T)])


@functools.partial(
    pl.kernel,
    out_type=jax.ShapeDtypeStruct((2, NPAD, DH), jnp.float32),
    mesh=_mesh,
    scratch_types=[
        pltpu.VMEM_SHARED((NPAD, DH), jnp.float32),
        pltpu.VMEM((RPS, EB), jnp.int32),
        pltpu.VMEM((RPS, EB), jnp.int32),
        pltpu.VMEM((EB, DH), jnp.float32),
        pltpu.SemaphoreType.DMA,
    ],
)
def _agg_kernel(g_hbm, src_hbm, dst_hbm, s_hbm, acc, src_v, dst_v, buf, sem):
    c = lax.axis_index("c")
    s = lax.axis_index("s")

    def _fill(r, carry):
        for k in range(DH // 16):
            buf[r, pl.ds(k * 16, 16)] = jnp.zeros((16,), jnp.float32)
        return carry

    lax.fori_loop(0, EB, _fill, 0)

    for k in range(4):
        pltpu.sync_copy(buf, acc.at[pl.ds(s * ROWS_T + k * EB, EB)])
    pltpu.sync_copy(buf.at[pl.ds(0, ROWS_T - 4 * EB)],
                    acc.at[pl.ds(s * ROWS_T + 4 * EB, ROWS_T - 4 * EB)])
    pltpu.sync_copy(src_hbm.at[pl.ds(s * RPS, RPS)], src_v)
    pltpu.sync_copy(dst_hbm.at[pl.ds(s * RPS, RPS)], dst_v)

    bias = c * N

    def _adjust(r, carry):
        for k in range(EB // 16):
            sl = (r, pl.ds(k * 16, 16))
            src_v[sl] = src_v[sl] + bias
        return carry

    lax.fori_loop(0, RPS, _adjust, 0)
    plsc.subcore_barrier()

    def _chunk(j, carry):
        pltpu.async_copy(g_hbm.at[src_v.at[j]], buf, sem).wait()
        pltpu.sync_copy(buf, acc.at[dst_v.at[j]], add=True)
        return carry

    lax.fori_loop(0, RPS, _chunk, 0)
    plsc.subcore_barrier()

    pltpu.sync_copy(acc.at[pl.ds(s * ROWS_T, ROWS_T)],
                    s_hbm.at[c, pl.ds(s * ROWS_T, ROWS_T)])


def _dinv_block(deg_ref):
    d = deg_ref[0, :, 0:1] + deg_ref[1, :, 0:1] + 1.0
    return lax.rsqrt(d)


def _store_halves(g, g_ref):
    g_ref[...] = jnp.stack([g[:, :DH], g[:, DH:]])


def _first_body(deg_ref, x_ref, w_ref, g_ref):
    dinv = _dinv_block(deg_ref)
    h = jnp.dot(x_ref[...], w_ref[...], preferred_element_type=jnp.float32)
    _store_halves(h * dinv, g_ref)


def _mid_body(deg_ref, s_ref, g_ref, b_ref, w_ref, o_ref):
    dinv = _dinv_block(deg_ref)
    t = jnp.concatenate([s_ref[0] + g_ref[0], s_ref[1] + g_ref[1]], axis=1)
    hh = jnp.maximum(t * dinv + b_ref[...][None, :], 0.0)
    h = jnp.dot(hh, w_ref[...], preferred_element_type=jnp.float32)
    _store_halves(h * dinv, o_ref)


def _last_body(deg_ref, s_ref, g_ref, b_ref, o_ref):
    dinv = _dinv_block(deg_ref)
    t = jnp.concatenate([s_ref[0] + g_ref[0], s_ref[1] + g_ref[1]], axis=1)
    o_ref[...] = jnp.maximum(t * dinv + b_ref[...][None, :], 0.0)


_deg_spec = pl.BlockSpec((2, BR, DEGW), lambda i: (0, i, 0))
_stack_spec = pl.BlockSpec((2, BR, DH), lambda i: (0, i, 0))
_full_spec = pl.BlockSpec((BR, D), lambda i: (i, 0))
_w_spec = pl.BlockSpec((D, D), lambda i: (0, 0))
_b_spec = pl.BlockSpec((D,), lambda i: (0,))

_g_shape = jax.ShapeDtypeStruct((2, N, DH), jnp.float32)

_first_tc = pl.pallas_call(
    _first_body,
    grid=(NB,),
    in_specs=[_deg_spec, _full_spec, _w_spec],
    out_specs=_stack_spec,
    out_shape=_g_shape,
)

_mid_tc = pl.pallas_call(
    _mid_body,
    grid=(NB,),
    in_specs=[_deg_spec, _stack_spec, _stack_spec, _b_spec, _w_spec],
    out_specs=_stack_spec,
    out_shape=_g_shape,
)

_last_tc = pl.pallas_call(
    _last_body,
    grid=(NB,),
    in_specs=[_deg_spec, _stack_spec, _stack_spec, _b_spec],
    out_specs=_full_spec,
    out_shape=jax.ShapeDtypeStruct((N, D), jnp.float32),
)


def kernel(x, edge_index, W0, b0, W1, b1, W2, b2):
    npad_e = EPAD - E
    src = jnp.concatenate(
        [edge_index[0].astype(jnp.int32),
         jnp.zeros((npad_e,), jnp.int32)]).reshape(ER, EB)
    dst = jnp.concatenate(
        [edge_index[1].astype(jnp.int32),
         N + (jnp.arange(npad_e, dtype=jnp.int32) % (NPAD - N))]).reshape(ER, EB)

    deg2 = _deg_kernel(dst)
    g = _first_tc(deg2, x, W0)
    s = _agg_kernel(g.reshape(2 * N, DH), src, dst)
    g = _mid_tc(deg2, s, g, b0, W1)
    s = _agg_kernel(g.reshape(2 * N, DH), src, dst)
    g = _mid_tc(deg2, s, g, b1, W2)
    s = _agg_kernel(g.reshape(2 * N, DH), src, dst)
    return _last_tc(deg2, s, g, b2)


# speedup vs baseline: 7.7666x; 1.2242x over previous
import functools

import jax
import jax.numpy as jnp
from jax import lax
from jax.experimental import pallas as pl
from jax.experimental.pallas import tpu as pltpu
from jax.experimental.pallas import tpu_sc as plsc

N = 10000
NPAD = 10112
ROWS_T = NPAD // 16
D = 256
DH = 128
E = 160000
EB = 128
EPAD = 163840
ER = EPAD // EB
RPS = ER // 16
RPS_DEG = ER // 32
DEGW = 16
NB = 25
BR = N // NB

_mesh = plsc.VectorSubcoreMesh(core_axis_name="c", subcore_axis_name="s")


def _unpack_chunk(packed_v, j, dst_u, src_u=None, bias=None):
    for k in range(EB // 16):
        sl = pl.ds(k * 16, 16)
        p = packed_v[j, sl]
        dst_u[sl] = lax.shift_right_logical(p, 16)
        if src_u is not None:
            src_u[sl] = (p & 0xFFFF) + bias


@functools.partial(
    pl.kernel,
    out_type=jax.ShapeDtypeStruct((2, NPAD, DEGW), jnp.float32),
    mesh=_mesh,
    scratch_types=[
        pltpu.VMEM_SHARED((NPAD, DEGW), jnp.float32),
        pltpu.VMEM((RPS_DEG, EB), jnp.int32),
        pltpu.VMEM((EB,), jnp.int32),
        pltpu.VMEM((EB, DEGW), jnp.float32),
        pltpu.VMEM((ROWS_T // 4, DEGW), jnp.float32),
    ],
)
def _deg_kernel(packed_hbm, deg_hbm, acc, packed_v, dst_u, ones_v, z_v):
    c = lax.axis_index("c")
    s = lax.axis_index("s")

    def _fill_z(r, carry):
        z_v[r, :] = jnp.zeros((DEGW,), jnp.float32)
        return carry

    def _fill_1(r, carry):
        ones_v[r, :] = jnp.ones((DEGW,), jnp.float32)
        return carry

    lax.fori_loop(0, ROWS_T // 4, _fill_z, 0)
    lax.fori_loop(0, EB, _fill_1, 0)

    for k in range(4):
        pltpu.sync_copy(z_v, acc.at[pl.ds(s * ROWS_T + k * (ROWS_T // 4), ROWS_T // 4)])
    pltpu.sync_copy(packed_hbm.at[pl.ds(c * (ER // 2) + s * RPS_DEG, RPS_DEG)],
                    packed_v)
    plsc.subcore_barrier()

    def _chunk(j, carry):
        _unpack_chunk(packed_v, j, dst_u)
        pltpu.sync_copy(ones_v, acc.at[dst_u], add=True)
        return carry

    lax.fori_loop(0, RPS_DEG, _chunk, 0)
    plsc.subcore_barrier()
    pltpu.sync_copy(acc.at[pl.ds(s * ROWS_T, ROWS_T)],
                    deg_hbm.at[c, pl.ds(s * ROWS_T, ROWS_T)])


@functools.partial(
    pl.kernel,
    out_type=jax.ShapeDtypeStruct((2, NPAD, DH), jnp.float32),
    mesh=_mesh,
    scratch_types=[
        pltpu.VMEM_SHARED((NPAD, DH), jnp.float32),
        pltpu.VMEM((RPS, EB), jnp.int32),
        pltpu.VMEM((EB, DH), jnp.float32),
        pltpu.VMEM((EB, DH), jnp.float32),
        pltpu.VMEM((EB,), jnp.int32),
        pltpu.VMEM((EB,), jnp.int32),
        pltpu.VMEM((EB,), jnp.int32),
        pltpu.VMEM((EB,), jnp.int32),
        pltpu.SemaphoreType.DMA,
        pltpu.SemaphoreType.DMA,
    ],
)
def _agg_kernel(g_hbm, packed_hbm, s_hbm, acc, packed_v,
                buf0, buf1, src_u0, src_u1, dst_u0, dst_u1, sem0, sem1):
    c = lax.axis_index("c")
    s = lax.axis_index("s")
    bias = c * N

    def _fill(r, carry):
        for k in range(DH // 16):
            buf0[r, pl.ds(k * 16, 16)] = jnp.zeros((16,), jnp.float32)
        return carry

    lax.fori_loop(0, EB, _fill, 0)

    for k in range(4):
        pltpu.sync_copy(buf0, acc.at[pl.ds(s * ROWS_T + k * EB, EB)])
    pltpu.sync_copy(buf0.at[pl.ds(0, ROWS_T - 4 * EB)],
                    acc.at[pl.ds(s * ROWS_T + 4 * EB, ROWS_T - 4 * EB)])
    pltpu.sync_copy(packed_hbm.at[pl.ds(s * RPS, RPS)], packed_v)
    plsc.subcore_barrier()

    def _gather0():
        return pltpu.make_async_copy(g_hbm.at[src_u0], buf0, sem0)

    def _gather1():
        return pltpu.make_async_copy(g_hbm.at[src_u1], buf1, sem1)

    _unpack_chunk(packed_v, 0, dst_u0, src_u0, bias)
    _gather0().start()

    def _pair(jj, carry):
        j = 2 * jj
        _unpack_chunk(packed_v, j + 1, dst_u1, src_u1, bias)
        _gather1().start()
        _gather0().wait()
        pltpu.sync_copy(buf0, acc.at[dst_u0], add=True)

        @pl.when(jj + 1 < RPS // 2)
        def _():
            _unpack_chunk(packed_v, j + 2, dst_u0, src_u0, bias)
            _gather0().start()

        _gather1().wait()
        pltpu.sync_copy(buf1, acc.at[dst_u1], add=True)
        return carry

    lax.fori_loop(0, RPS // 2, _pair, 0)
    plsc.subcore_barrier()

    pltpu.sync_copy(acc.at[pl.ds(s * ROWS_T, ROWS_T)],
                    s_hbm.at[c, pl.ds(s * ROWS_T, ROWS_T)])


def _dinv_block(deg_ref):
    d = deg_ref[0, :, 0:1] + deg_ref[1, :, 0:1] + 1.0
    return lax.rsqrt(d)


def _store_halves(g, g_ref):
    g_ref[...] = jnp.stack([g[:, :DH], g[:, DH:]])


def _first_body(deg_ref, x_ref, w_ref, g_ref):
    dinv = _dinv_block(deg_ref)
    h = jnp.dot(x_ref[...], w_ref[...], preferred_element_type=jnp.float32)
    _store_halves(h * dinv, g_ref)


def _mid_body(deg_ref, s_ref, g_ref, b_ref, w_ref, o_ref):
    dinv = _dinv_block(deg_ref)
    t = jnp.concatenate([s_ref[0] + g_ref[0], s_ref[1] + g_ref[1]], axis=1)
    hh = jnp.maximum(t * dinv + b_ref[...][None, :], 0.0)
    h = jnp.dot(hh, w_ref[...], preferred_element_type=jnp.float32)
    _store_halves(h * dinv, o_ref)


def _last_body(deg_ref, s_ref, g_ref, b_ref, o_ref):
    dinv = _dinv_block(deg_ref)
    t = jnp.concatenate([s_ref[0] + g_ref[0], s_ref[1] + g_ref[1]], axis=1)
    o_ref[...] = jnp.maximum(t * dinv + b_ref[...][None, :], 0.0)


_deg_spec = pl.BlockSpec((2, BR, DEGW), lambda i: (0, i, 0))
_stack_spec = pl.BlockSpec((2, BR, DH), lambda i: (0, i, 0))
_full_spec = pl.BlockSpec((BR, D), lambda i: (i, 0))
_w_spec = pl.BlockSpec((D, D), lambda i: (0, 0))
_b_spec = pl.BlockSpec((D,), lambda i: (0,))

_g_shape = jax.ShapeDtypeStruct((2, N, DH), jnp.float32)

_first_tc = pl.pallas_call(
    _first_body,
    grid=(NB,),
    in_specs=[_deg_spec, _full_spec, _w_spec],
    out_specs=_stack_spec,
    out_shape=_g_shape,
)

_mid_tc = pl.pallas_call(
    _mid_body,
    grid=(NB,),
    in_specs=[_deg_spec, _stack_spec, _stack_spec, _b_spec, _w_spec],
    out_specs=_stack_spec,
    out_shape=_g_shape,
)

_last_tc = pl.pallas_call(
    _last_body,
    grid=(NB,),
    in_specs=[_deg_spec, _stack_spec, _stack_spec, _b_spec],
    out_specs=_full_spec,
    out_shape=jax.ShapeDtypeStruct((N, D), jnp.float32),
)


def kernel(x, edge_index, W0, b0, W1, b1, W2, b2):
    npad_e = EPAD - E
    src = jnp.concatenate(
        [edge_index[0].astype(jnp.int32), jnp.zeros((npad_e,), jnp.int32)])
    dst = jnp.concatenate(
        [edge_index[1].astype(jnp.int32),
         N + (jnp.arange(npad_e, dtype=jnp.int32) % (NPAD - N))])
    packed = (src | (dst << 16)).reshape(ER, EB)

    deg2 = _deg_kernel(packed)
    g = _first_tc(deg2, x, W0)
    s = _agg_kernel(g.reshape(2 * N, DH), packed)
    g = _mid_tc(deg2, s, g, b0, W1)
    s = _agg_kernel(g.reshape(2 * N, DH), packed)
    g = _mid_tc(deg2, s, g, b1, W2)
    s = _agg_kernel(g.reshape(2 * N, DH), packed)
    return _last_tc(deg2, s, g, b2)


# speedup vs baseline: 7.7830x; 1.0021x over previous
import functools

import jax
import jax.numpy as jnp
from jax import lax
from jax.experimental import pallas as pl
from jax.experimental.pallas import tpu as pltpu
from jax.experimental.pallas import tpu_sc as plsc

N = 10000
NPAD = 10112
ROWS_T = NPAD // 16
D = 256
DH = 128
E = 160000
EB = 128
EPAD = 163840
ER = EPAD // EB
RPS = ER // 16
RPS_DEG = ER // 32
DEGW = 16
NB = 25
BR = N // NB

_mesh = plsc.VectorSubcoreMesh(core_axis_name="c", subcore_axis_name="s")


def _unpack_chunk(packed_v, j, dst_u, src_u=None, bias=None):
    for k in range(EB // 16):
        sl = pl.ds(k * 16, 16)
        p = packed_v[j, sl]
        dst_u[sl] = lax.shift_right_logical(p, 16)
        if src_u is not None:
            src_u[sl] = (p & 0xFFFF) + bias


@functools.partial(
    pl.kernel,
    out_type=jax.ShapeDtypeStruct((2, NPAD, DEGW), jnp.float32),
    mesh=_mesh,
    scratch_types=[
        pltpu.VMEM_SHARED((NPAD, DEGW), jnp.float32),
        pltpu.VMEM((RPS_DEG, EB), jnp.int32),
        pltpu.VMEM((EB,), jnp.int32),
        pltpu.VMEM((EB, DEGW), jnp.float32),
        pltpu.VMEM((ROWS_T // 4, DEGW), jnp.float32),
    ],
)
def _deg_kernel(packed_hbm, deg_hbm, acc, packed_v, dst_u, ones_v, z_v):
    c = lax.axis_index("c")
    s = lax.axis_index("s")

    def _fill_z(r, carry):
        z_v[r, :] = jnp.zeros((DEGW,), jnp.float32)
        return carry

    def _fill_1(r, carry):
        ones_v[r, :] = jnp.ones((DEGW,), jnp.float32)
        return carry

    lax.fori_loop(0, ROWS_T // 4, _fill_z, 0)
    lax.fori_loop(0, EB, _fill_1, 0)

    for k in range(4):
        pltpu.sync_copy(z_v, acc.at[pl.ds(s * ROWS_T + k * (ROWS_T // 4), ROWS_T // 4)])
    pltpu.sync_copy(packed_hbm.at[pl.ds(c * (ER // 2) + s * RPS_DEG, RPS_DEG)],
                    packed_v)
    plsc.subcore_barrier()

    def _chunk(j, carry):
        _unpack_chunk(packed_v, j, dst_u)
        pltpu.sync_copy(ones_v, acc.at[dst_u], add=True)
        return carry

    lax.fori_loop(0, RPS_DEG, _chunk, 0)
    plsc.subcore_barrier()
    pltpu.sync_copy(acc.at[pl.ds(s * ROWS_T, ROWS_T)],
                    deg_hbm.at[c, pl.ds(s * ROWS_T, ROWS_T)])


@functools.partial(
    pl.kernel,
    out_type=jax.ShapeDtypeStruct((2, NPAD, DH), jnp.float32),
    mesh=_mesh,
    scratch_types=[
        pltpu.VMEM_SHARED((NPAD, DH), jnp.float32),
        pltpu.VMEM((RPS, EB), jnp.int32),
        pltpu.VMEM((EB, DH), jnp.float32),
        pltpu.VMEM((EB, DH), jnp.float32),
        pltpu.VMEM((EB,), jnp.int32),
        pltpu.VMEM((EB,), jnp.int32),
        pltpu.VMEM((EB,), jnp.int32),
        pltpu.VMEM((EB,), jnp.int32),
        pltpu.SemaphoreType.DMA,
        pltpu.SemaphoreType.DMA,
    ],
)
def _agg_kernel(g_hbm, packed_hbm, s_hbm, acc, packed_v,
                buf0, buf1, src_u0, src_u1, dst_u0, dst_u1, sem0, sem1):
    c = lax.axis_index("c")
    s = lax.axis_index("s")
    bias = c * N

    def _fill(r, carry):
        for k in range(DH // 16):
            buf0[r, pl.ds(k * 16, 16)] = jnp.zeros((16,), jnp.float32)
        return carry

    lax.fori_loop(0, EB, _fill, 0)

    for k in range(4):
        pltpu.sync_copy(buf0, acc.at[pl.ds(s * ROWS_T + k * EB, EB)])
    pltpu.sync_copy(buf0.at[pl.ds(0, ROWS_T - 4 * EB)],
                    acc.at[pl.ds(s * ROWS_T + 4 * EB, ROWS_T - 4 * EB)])
    pltpu.sync_copy(packed_hbm.at[pl.ds(s * RPS, RPS)], packed_v)
    plsc.subcore_barrier()

    NSPLIT = 4
    H = EB // NSPLIT

    def _descs(src_u, buf, sem):
        return [pltpu.make_async_copy(g_hbm.at[src_u.at[pl.ds(k * H, H)]],
                                      buf.at[pl.ds(k * H, H)], sem)
                for k in range(NSPLIT)]

    class _G:
        def __init__(self, src_u, buf, sem):
            self.args = (src_u, buf, sem)

        def start(self):
            for d in _descs(*self.args):
                d.start()

        def wait(self):
            for d in _descs(*self.args):
                d.wait()

    def _gather0():
        return _G(src_u0, buf0, sem0)

    def _gather1():
        return _G(src_u1, buf1, sem1)

    _unpack_chunk(packed_v, 0, dst_u0, src_u0, bias)
    _gather0().start()

    def _pair(jj, carry):
        j = 2 * jj
        _unpack_chunk(packed_v, j + 1, dst_u1, src_u1, bias)
        _gather1().start()
        _gather0().wait()
        pltpu.sync_copy(buf0, acc.at[dst_u0], add=True)

        @pl.when(jj + 1 < RPS // 2)
        def _():
            _unpack_chunk(packed_v, j + 2, dst_u0, src_u0, bias)
            _gather0().start()

        _gather1().wait()
        pltpu.sync_copy(buf1, acc.at[dst_u1], add=True)
        return carry

    lax.fori_loop(0, RPS // 2, _pair, 0)
    plsc.subcore_barrier()

    pltpu.sync_copy(acc.at[pl.ds(s * ROWS_T, ROWS_T)],
                    s_hbm.at[c, pl.ds(s * ROWS_T, ROWS_T)])


def _dinv_block(deg_ref):
    d = deg_ref[0, :, 0:1] + deg_ref[1, :, 0:1] + 1.0
    return lax.rsqrt(d)


def _store_halves(g, g_ref):
    g_ref[...] = jnp.stack([g[:, :DH], g[:, DH:]])


def _first_body(deg_ref, x_ref, w_ref, g_ref):
    dinv = _dinv_block(deg_ref)
    h = jnp.dot(x_ref[...], w_ref[...], preferred_element_type=jnp.float32)
    _store_halves(h * dinv, g_ref)


def _mid_body(deg_ref, s_ref, g_ref, b_ref, w_ref, o_ref):
    dinv = _dinv_block(deg_ref)
    t = jnp.concatenate([s_ref[0] + g_ref[0], s_ref[1] + g_ref[1]], axis=1)
    hh = jnp.maximum(t * dinv + b_ref[...][None, :], 0.0)
    h = jnp.dot(hh, w_ref[...], preferred_element_type=jnp.float32)
    _store_halves(h * dinv, o_ref)


def _last_body(deg_ref, s_ref, g_ref, b_ref, o_ref):
    dinv = _dinv_block(deg_ref)
    t = jnp.concatenate([s_ref[0] + g_ref[0], s_ref[1] + g_ref[1]], axis=1)
    o_ref[...] = jnp.maximum(t * dinv + b_ref[...][None, :], 0.0)


_deg_spec = pl.BlockSpec((2, BR, DEGW), lambda i: (0, i, 0))
_stack_spec = pl.BlockSpec((2, BR, DH), lambda i: (0, i, 0))
_full_spec = pl.BlockSpec((BR, D), lambda i: (i, 0))
_w_spec = pl.BlockSpec((D, D), lambda i: (0, 0))
_b_spec = pl.BlockSpec((D,), lambda i: (0,))

_g_shape = jax.ShapeDtypeStruct((2, N, DH), jnp.float32)

_first_tc = pl.pallas_call(
    _first_body,
    grid=(NB,),
    in_specs=[_deg_spec, _full_spec, _w_spec],
    out_specs=_stack_spec,
    out_shape=_g_shape,
)

_mid_tc = pl.pallas_call(
    _mid_body,
    grid=(NB,),
    in_specs=[_deg_spec, _stack_spec, _stack_spec, _b_spec, _w_spec],
    out_specs=_stack_spec,
    out_shape=_g_shape,
)

_last_tc = pl.pallas_call(
    _last_body,
    grid=(NB,),
    in_specs=[_deg_spec, _stack_spec, _stack_spec, _b_spec],
    out_specs=_full_spec,
    out_shape=jax.ShapeDtypeStruct((N, D), jnp.float32),
)


def kernel(x, edge_index, W0, b0, W1, b1, W2, b2):
    npad_e = EPAD - E
    src = jnp.concatenate(
        [edge_index[0].astype(jnp.int32), jnp.zeros((npad_e,), jnp.int32)])
    dst = jnp.concatenate(
        [edge_index[1].astype(jnp.int32),
         N + (jnp.arange(npad_e, dtype=jnp.int32) % (NPAD - N))])
    packed = (src | (dst << 16)).reshape(ER, EB)

    deg2 = _deg_kernel(packed)
    g = _first_tc(deg2, x, W0)
    s = _agg_kernel(g.reshape(2 * N, DH), packed)
    g = _mid_tc(deg2, s, g, b0, W1)
    s = _agg_kernel(g.reshape(2 * N, DH), packed)
    g = _mid_tc(deg2, s, g, b1, W2)
    s = _agg_kernel(g.reshape(2 * N, DH), packed)
    return _last_tc(deg2, s, g, b2)


# speedup vs baseline: 10.4817x; 1.3467x over previous
import functools

import jax
import jax.numpy as jnp
from jax import lax
from jax.experimental import pallas as pl
from jax.experimental.pallas import tpu as pltpu
from jax.experimental.pallas import tpu_sc as plsc

N = 10000
NPAD = 10112
ROWS_T = NPAD // 16
D = 256
DH = 128
DQ = 64
E = 160000
EB = 128
EPAD = 163840
ER = EPAD // EB
RPS = ER // 16
RPS_DEG = ER // 32
DEGW = 16
NB = 25
BR = N // NB

_mesh = plsc.VectorSubcoreMesh(core_axis_name="c", subcore_axis_name="s")


def _unpack_chunk(packed_v, j, dst_u, src_u=None, bias=None):
    for k in range(EB // 16):
        sl = pl.ds(k * 16, 16)
        p = packed_v[j, sl]
        dst_u[sl] = lax.shift_right_logical(p, 16)
        if src_u is not None:
            src_u[sl] = (p & 0xFFFF) + bias


@functools.partial(
    pl.kernel,
    out_type=jax.ShapeDtypeStruct((2, NPAD, DEGW), jnp.float32),
    mesh=_mesh,
    scratch_types=[
        pltpu.VMEM_SHARED((NPAD, DEGW), jnp.float32),
        pltpu.VMEM((RPS_DEG, EB), jnp.int32),
        pltpu.VMEM((EB,), jnp.int32),
        pltpu.VMEM((EB, DEGW), jnp.float32),
        pltpu.VMEM((ROWS_T // 4, DEGW), jnp.float32),
    ],
)
def _deg_kernel(packed_hbm, deg_hbm, acc, packed_v, dst_u, ones_v, z_v):
    c = lax.axis_index("c")
    s = lax.axis_index("s")

    def _fill_z(r, carry):
        z_v[r, :] = jnp.zeros((DEGW,), jnp.float32)
        return carry

    def _fill_1(r, carry):
        ones_v[r, :] = jnp.ones((DEGW,), jnp.float32)
        return carry

    lax.fori_loop(0, ROWS_T // 4, _fill_z, 0)
    lax.fori_loop(0, EB, _fill_1, 0)

    for k in range(4):
        pltpu.sync_copy(z_v, acc.at[pl.ds(s * ROWS_T + k * (ROWS_T // 4), ROWS_T // 4)])
    pltpu.sync_copy(packed_hbm.at[pl.ds(c * (ER // 2) + s * RPS_DEG, RPS_DEG)],
                    packed_v)
    plsc.subcore_barrier()

    def _chunk(j, carry):
        _unpack_chunk(packed_v, j, dst_u)
        pltpu.sync_copy(ones_v, acc.at[dst_u], add=True)
        return carry

    lax.fori_loop(0, RPS_DEG, _chunk, 0)
    plsc.subcore_barrier()
    pltpu.sync_copy(acc.at[pl.ds(s * ROWS_T, ROWS_T)],
                    deg_hbm.at[c, pl.ds(s * ROWS_T, ROWS_T)])


@functools.partial(
    pl.kernel,
    out_type=jax.ShapeDtypeStruct((4, NPAD, DQ), jnp.float32),
    mesh=_mesh,
    compiler_params=pltpu.CompilerParams(use_tc_tiling_on_sc=False),
    scratch_types=[
        pltpu.VMEM_SHARED((NPAD, DQ), jnp.float32),
        pltpu.VMEM_SHARED((NPAD, DQ), jnp.float32),
        pltpu.VMEM((RPS, EB), jnp.int32),
        pltpu.VMEM((EB, DQ), jnp.float32),
        pltpu.VMEM((EB, DQ), jnp.float32),
        pltpu.VMEM((EB,), jnp.int32),
        pltpu.VMEM((EB,), jnp.int32),
        pltpu.VMEM((EB,), jnp.int32),
        pltpu.VMEM((EB,), jnp.int32),
        pltpu.SemaphoreType.DMA,
        pltpu.SemaphoreType.DMA,
    ],
)
def _agg_kernel(g_hbm, packed_hbm, s_hbm, acc, gst, packed_v,
                buf0, buf1, src_u0, src_u1, dst_u0, dst_u1, sem0, sem1):
    c = lax.axis_index("c")
    s = lax.axis_index("s")
    rows = pl.ds(s * ROWS_T, ROWS_T)
    pltpu.sync_copy(packed_hbm.at[pl.ds(s * RPS, RPS)], packed_v)

    for p in range(2):
        q = 2 * c + p

        def _fill(r, carry):
            for k in range(DQ // 16):
                buf0[r, pl.ds(k * 16, 16)] = jnp.zeros((16,), jnp.float32)
            return carry

        lax.fori_loop(0, EB, _fill, 0)

        pltpu.sync_copy(g_hbm.at[q, rows], gst.at[rows])
        for k in range(4):
            pltpu.sync_copy(buf0, acc.at[pl.ds(s * ROWS_T + k * EB, EB)])
        pltpu.sync_copy(buf0.at[pl.ds(0, ROWS_T - 4 * EB)],
                        acc.at[pl.ds(s * ROWS_T + 4 * EB, ROWS_T - 4 * EB)])
        plsc.subcore_barrier()

        NSPLIT = 2
        H = EB // NSPLIT

        def _descs(src_u, buf, sem):
            return [pltpu.make_async_copy(gst.at[src_u.at[pl.ds(k * H, H)]],
                                          buf.at[pl.ds(k * H, H)], sem)
                    for k in range(NSPLIT)]

        def _start(src_u, buf, sem):
            for d in _descs(src_u, buf, sem):
                d.start()

        def _wait(src_u, buf, sem):
            for d in _descs(src_u, buf, sem):
                d.wait()

        _unpack_chunk(packed_v, 0, dst_u0, src_u0, 0)
        _start(src_u0, buf0, sem0)

        def _pair(jj, carry):
            j = 2 * jj
            _unpack_chunk(packed_v, j + 1, dst_u1, src_u1, 0)
            _start(src_u1, buf1, sem1)
            _wait(src_u0, buf0, sem0)
            pltpu.sync_copy(buf0, acc.at[dst_u0], add=True)

            @pl.when(jj + 1 < RPS // 2)
            def _():
                _unpack_chunk(packed_v, j + 2, dst_u0, src_u0, 0)
                _start(src_u0, buf0, sem0)

            _wait(src_u1, buf1, sem1)
            pltpu.sync_copy(buf1, acc.at[dst_u1], add=True)
            return carry

        lax.fori_loop(0, RPS // 2, _pair, 0)
        plsc.subcore_barrier()

        pltpu.sync_copy(acc.at[rows], s_hbm.at[q, rows])


def _dinv_block(deg_ref):
    d = deg_ref[0, :, 0:1] + deg_ref[1, :, 0:1] + 1.0
    return lax.rsqrt(d)


def _store_quarters(g, g_ref):
    g_ref[...] = jnp.stack([g[:, q * DQ:(q + 1) * DQ] for q in range(4)])


def _cat_quarters(s_ref, g_ref):
    return jnp.concatenate([s_ref[q] + g_ref[q] for q in range(4)], axis=1)


def _first_body(deg_ref, x_ref, w_ref, g_ref):
    dinv = _dinv_block(deg_ref)
    h = jnp.dot(x_ref[...], w_ref[...], preferred_element_type=jnp.float32)
    _store_quarters(h * dinv, g_ref)


def _mid_body(deg_ref, s_ref, g_ref, b_ref, w_ref, o_ref):
    dinv = _dinv_block(deg_ref)
    t = _cat_quarters(s_ref, g_ref)
    hh = jnp.maximum(t * dinv + b_ref[...][None, :], 0.0)
    h = jnp.dot(hh, w_ref[...], preferred_element_type=jnp.float32)
    _store_quarters(h * dinv, o_ref)


def _last_body(deg_ref, s_ref, g_ref, b_ref, o_ref):
    dinv = _dinv_block(deg_ref)
    t = _cat_quarters(s_ref, g_ref)
    o_ref[...] = jnp.maximum(t * dinv + b_ref[...][None, :], 0.0)


_deg_spec = pl.BlockSpec((2, BR, DEGW), lambda i: (0, i, 0))
_stack_spec = pl.BlockSpec((4, BR, DQ), lambda i: (0, i, 0))
_full_spec = pl.BlockSpec((BR, D), lambda i: (i, 0))
_w_spec = pl.BlockSpec((D, D), lambda i: (0, 0))
_b_spec = pl.BlockSpec((D,), lambda i: (0,))

_g_shape = jax.ShapeDtypeStruct((4, NPAD, DQ), jnp.float32)

_first_tc = pl.pallas_call(
    _first_body,
    grid=(NB,),
    in_specs=[_deg_spec, _full_spec, _w_spec],
    out_specs=_stack_spec,
    out_shape=_g_shape,
)

_mid_tc = pl.pallas_call(
    _mid_body,
    grid=(NB,),
    in_specs=[_deg_spec, _stack_spec, _stack_spec, _b_spec, _w_spec],
    out_specs=_stack_spec,
    out_shape=_g_shape,
)

_last_tc = pl.pallas_call(
    _last_body,
    grid=(NB,),
    in_specs=[_deg_spec, _stack_spec, _stack_spec, _b_spec],
    out_specs=_full_spec,
    out_shape=jax.ShapeDtypeStruct((N, D), jnp.float32),
)


def kernel(x, edge_index, W0, b0, W1, b1, W2, b2):
    npad_e = EPAD - E
    src = jnp.concatenate(
        [edge_index[0].astype(jnp.int32), jnp.zeros((npad_e,), jnp.int32)])
    dst = jnp.concatenate(
        [edge_index[1].astype(jnp.int32),
         N + (jnp.arange(npad_e, dtype=jnp.int32) % (NPAD - N))])
    packed = (src | (dst << 16)).reshape(ER, EB)

    deg2 = _deg_kernel(packed)
    g = _first_tc(deg2, x, W0)
    s = _agg_kernel(g, packed)
    g = _mid_tc(deg2, s, g, b0, W1)
    s = _agg_kernel(g, packed)
    g = _mid_tc(deg2, s, g, b1, W2)
    s = _agg_kernel(g, packed)
    return _last_tc(deg2, s, g, b2)


# speedup vs baseline: 11.1873x; 1.0673x over previous
import functools

import jax
import jax.numpy as jnp
from jax import lax
from jax.experimental import pallas as pl
from jax.experimental.pallas import tpu as pltpu
from jax.experimental.pallas import tpu_sc as plsc

N = 10000
NPAD = 10112
ROWS_T = NPAD // 16
D = 256
DH = 128
DQ = 64
E = 160000
EB = 128
EPAD = 163840
ER = EPAD // EB
RPS = ER // 16
RPS_DEG = ER // 32
DEGW = 16
NB = 25
BR = N // NB

_mesh = plsc.VectorSubcoreMesh(core_axis_name="c", subcore_axis_name="s")


def _unpack_chunk(packed_v, j, dst_u, src_u=None, bias=None):
    for k in range(EB // 16):
        sl = pl.ds(k * 16, 16)
        p = packed_v[j, sl]
        dst_u[sl] = lax.shift_right_logical(p, 16)
        if src_u is not None:
            src_u[sl] = (p & 0xFFFF) + bias


@functools.partial(
    pl.kernel,
    out_type=jax.ShapeDtypeStruct((2, NPAD, DEGW), jnp.float32),
    mesh=_mesh,
    scratch_types=[
        pltpu.VMEM_SHARED((NPAD, DEGW), jnp.float32),
        pltpu.VMEM((RPS_DEG, EB), jnp.int32),
        pltpu.VMEM((EB,), jnp.int32),
        pltpu.VMEM((EB, DEGW), jnp.float32),
        pltpu.VMEM((ROWS_T // 4, DEGW), jnp.float32),
    ],
)
def _deg_kernel(packed_hbm, deg_hbm, acc, packed_v, dst_u, ones_v, z_v):
    c = lax.axis_index("c")
    s = lax.axis_index("s")

    def _fill_z(r, carry):
        z_v[r, :] = jnp.zeros((DEGW,), jnp.float32)
        return carry

    def _fill_1(r, carry):
        ones_v[r, :] = jnp.ones((DEGW,), jnp.float32)
        return carry

    lax.fori_loop(0, ROWS_T // 4, _fill_z, 0)
    lax.fori_loop(0, EB, _fill_1, 0)

    for k in range(4):
        pltpu.sync_copy(z_v, acc.at[pl.ds(s * ROWS_T + k * (ROWS_T // 4), ROWS_T // 4)])
    pltpu.sync_copy(packed_hbm.at[pl.ds(c * (ER // 2) + s * RPS_DEG, RPS_DEG)],
                    packed_v)
    plsc.subcore_barrier()

    def _chunk(j, carry):
        _unpack_chunk(packed_v, j, dst_u)
        pltpu.sync_copy(ones_v, acc.at[dst_u], add=True)
        return carry

    lax.fori_loop(0, RPS_DEG, _chunk, 0)
    plsc.subcore_barrier()
    pltpu.sync_copy(acc.at[pl.ds(s * ROWS_T, ROWS_T)],
                    deg_hbm.at[c, pl.ds(s * ROWS_T, ROWS_T)])


@functools.partial(
    pl.kernel,
    out_type=jax.ShapeDtypeStruct((NPAD, D), jnp.float32),
    mesh=_mesh,
    compiler_params=pltpu.CompilerParams(use_tc_tiling_on_sc=False),
    scratch_types=[
        pltpu.VMEM_SHARED((NPAD, DQ), jnp.float32),
        pltpu.VMEM_SHARED((NPAD, DQ), jnp.float32),
        pltpu.VMEM((RPS, EB), jnp.int32),
        pltpu.VMEM((EB, DQ), jnp.float32),
        pltpu.VMEM((EB, DQ), jnp.float32),
        pltpu.VMEM((EB,), jnp.int32),
        pltpu.VMEM((EB,), jnp.int32),
        pltpu.VMEM((EB,), jnp.int32),
        pltpu.VMEM((EB,), jnp.int32),
        pltpu.SemaphoreType.DMA,
        pltpu.SemaphoreType.DMA,
    ],
)
def _agg_kernel(g_hbm, packed_hbm, s_hbm, acc, gst, packed_v,
                buf0, buf1, src_u0, src_u1, dst_u0, dst_u1, sem0, sem1):
    c = lax.axis_index("c")
    s = lax.axis_index("s")
    rows = pl.ds(s * ROWS_T, ROWS_T)
    pltpu.sync_copy(packed_hbm.at[pl.ds(s * RPS, RPS)], packed_v)

    for p in range(2):
        q = 2 * c + p

        def _fill(r, carry):
            for k in range(DQ // 16):
                buf0[r, pl.ds(k * 16, 16)] = jnp.zeros((16,), jnp.float32)
            return carry

        lax.fori_loop(0, EB, _fill, 0)

        pltpu.sync_copy(g_hbm.at[rows, pl.ds(q * DQ, DQ)], gst.at[rows])
        for k in range(4):
            pltpu.sync_copy(buf0, acc.at[pl.ds(s * ROWS_T + k * EB, EB)])
        pltpu.sync_copy(buf0.at[pl.ds(0, ROWS_T - 4 * EB)],
                        acc.at[pl.ds(s * ROWS_T + 4 * EB, ROWS_T - 4 * EB)])
        plsc.subcore_barrier()

        NSPLIT = 2
        H = EB // NSPLIT

        def _descs(src_u, buf, sem):
            return [pltpu.make_async_copy(gst.at[src_u.at[pl.ds(k * H, H)]],
                                          buf.at[pl.ds(k * H, H)], sem)
                    for k in range(NSPLIT)]

        def _start(src_u, buf, sem):
            for d in _descs(src_u, buf, sem):
                d.start()

        def _wait(src_u, buf, sem):
            for d in _descs(src_u, buf, sem):
                d.wait()

        _unpack_chunk(packed_v, 0, dst_u0, src_u0, 0)
        _start(src_u0, buf0, sem0)

        def _pair(jj, carry):
            j = 2 * jj
            _unpack_chunk(packed_v, j + 1, dst_u1, src_u1, 0)
            _start(src_u1, buf1, sem1)
            _wait(src_u0, buf0, sem0)
            pltpu.sync_copy(buf0, acc.at[dst_u0], add=True)

            @pl.when(jj + 1 < RPS // 2)
            def _():
                _unpack_chunk(packed_v, j + 2, dst_u0, src_u0, 0)
                _start(src_u0, buf0, sem0)

            _wait(src_u1, buf1, sem1)
            pltpu.sync_copy(buf1, acc.at[dst_u1], add=True)
            return carry

        lax.fori_loop(0, RPS // 2, _pair, 0)
        plsc.subcore_barrier()

        pltpu.sync_copy(acc.at[rows], s_hbm.at[rows, pl.ds(q * DQ, DQ)])


def _dinv_block(deg_ref):
    d = deg_ref[0, :, 0:1] + deg_ref[1, :, 0:1] + 1.0
    return lax.rsqrt(d)


def _first_body(deg_ref, x_ref, w_ref, g_ref):
    dinv = _dinv_block(deg_ref)
    h = jnp.dot(x_ref[...], w_ref[...], preferred_element_type=jnp.float32)
    g_ref[...] = h * dinv


def _mid_body(deg_ref, s_ref, g_ref, b_ref, w_ref, o_ref):
    dinv = _dinv_block(deg_ref)
    t = s_ref[...] + g_ref[...]
    hh = jnp.maximum(t * dinv + b_ref[...][None, :], 0.0)
    h = jnp.dot(hh, w_ref[...], preferred_element_type=jnp.float32)
    o_ref[...] = h * dinv


def _last_body(deg_ref, s_ref, g_ref, b_ref, o_ref):
    dinv = _dinv_block(deg_ref)
    t = s_ref[...] + g_ref[...]
    o_ref[...] = jnp.maximum(t * dinv + b_ref[...][None, :], 0.0)


_deg_spec = pl.BlockSpec((2, BR, DEGW), lambda i: (0, i, 0))
_full_spec = pl.BlockSpec((BR, D), lambda i: (i, 0))
_w_spec = pl.BlockSpec((D, D), lambda i: (0, 0))
_b_spec = pl.BlockSpec((D,), lambda i: (0,))

_g_shape = jax.ShapeDtypeStruct((NPAD, D), jnp.float32)

_first_tc = pl.pallas_call(
    _first_body,
    grid=(NB,),
    in_specs=[_deg_spec, _full_spec, _w_spec],
    out_specs=_full_spec,
    out_shape=_g_shape,
)

_mid_tc = pl.pallas_call(
    _mid_body,
    grid=(NB,),
    in_specs=[_deg_spec, _full_spec, _full_spec, _b_spec, _w_spec],
    out_specs=_full_spec,
    out_shape=_g_shape,
)

_last_tc = pl.pallas_call(
    _last_body,
    grid=(NB,),
    in_specs=[_deg_spec, _full_spec, _full_spec, _b_spec],
    out_specs=_full_spec,
    out_shape=jax.ShapeDtypeStruct((N, D), jnp.float32),
)


def kernel(x, edge_index, W0, b0, W1, b1, W2, b2):
    npad_e = EPAD - E
    src = jnp.concatenate(
        [edge_index[0].astype(jnp.int32), jnp.zeros((npad_e,), jnp.int32)])
    dst = jnp.concatenate(
        [edge_index[1].astype(jnp.int32),
         N + (jnp.arange(npad_e, dtype=jnp.int32) % (NPAD - N))])
    packed = (src | (dst << 16)).reshape(ER, EB)

    deg2 = _deg_kernel(packed)
    g = _first_tc(deg2, x, W0)
    s = _agg_kernel(g, packed)
    g = _mid_tc(deg2, s, g, b0, W1)
    s = _agg_kernel(g, packed)
    g = _mid_tc(deg2, s, g, b1, W2)
    s = _agg_kernel(g, packed)
    return _last_tc(deg2, s, g, b2)


# speedup vs baseline: 11.2182x; 1.0028x over previous
import functools

import jax
import jax.numpy as jnp
from jax import lax
from jax.experimental import pallas as pl
from jax.experimental.pallas import tpu as pltpu
from jax.experimental.pallas import tpu_sc as plsc

N = 10000
NPAD = 10112
ROWS_T = NPAD // 16
D = 256
DH = 128
DQ = 64
E = 160000
EB = 128
EPAD = 163840
ER = EPAD // EB
RPS = ER // 16
RPS_DEG = ER // 32
DEGW = 16
NB = 25
BR = N // NB

_mesh = plsc.VectorSubcoreMesh(core_axis_name="c", subcore_axis_name="s")


def _unpack_chunk(packed_v, j, dst_u, src_u=None, bias=None):
    for k in range(EB // 16):
        sl = pl.ds(k * 16, 16)
        p = packed_v[j, sl]
        dst_u[sl] = lax.shift_right_logical(p, 16)
        if src_u is not None:
            src_u[sl] = (p & 0xFFFF) + bias


@functools.partial(
    pl.kernel,
    out_type=jax.ShapeDtypeStruct((2, NPAD, DEGW), jnp.float32),
    mesh=_mesh,
    scratch_types=[
        pltpu.VMEM_SHARED((NPAD, DEGW), jnp.float32),
        pltpu.VMEM((RPS_DEG, EB), jnp.int32),
        pltpu.VMEM((EB,), jnp.int32),
        pltpu.VMEM((EB, DEGW), jnp.float32),
        pltpu.VMEM((ROWS_T // 4, DEGW), jnp.float32),
    ],
)
def _deg_kernel(packed_hbm, deg_hbm, acc, packed_v, dst_u, ones_v, z_v):
    c = lax.axis_index("c")
    s = lax.axis_index("s")

    def _fill_z(r, carry):
        z_v[r, :] = jnp.zeros((DEGW,), jnp.float32)
        return carry

    def _fill_1(r, carry):
        ones_v[r, :] = jnp.ones((DEGW,), jnp.float32)
        return carry

    lax.fori_loop(0, ROWS_T // 4, _fill_z, 0)
    lax.fori_loop(0, EB, _fill_1, 0)

    for k in range(4):
        pltpu.sync_copy(z_v, acc.at[pl.ds(s * ROWS_T + k * (ROWS_T // 4), ROWS_T // 4)])
    pltpu.sync_copy(packed_hbm.at[pl.ds(c * (ER // 2) + s * RPS_DEG, RPS_DEG)],
                    packed_v)
    plsc.subcore_barrier()

    def _chunk(j, carry):
        _unpack_chunk(packed_v, j, dst_u)
        pltpu.sync_copy(ones_v, acc.at[dst_u], add=True)
        return carry

    lax.fori_loop(0, RPS_DEG, _chunk, 0)
    plsc.subcore_barrier()
    pltpu.sync_copy(acc.at[pl.ds(s * ROWS_T, ROWS_T)],
                    deg_hbm.at[c, pl.ds(s * ROWS_T, ROWS_T)])


@functools.partial(
    pl.kernel,
    out_type=jax.ShapeDtypeStruct((NPAD, D), jnp.float32),
    mesh=_mesh,
    compiler_params=pltpu.CompilerParams(use_tc_tiling_on_sc=False),
    scratch_types=[
        pltpu.VMEM_SHARED((NPAD, DQ), jnp.float32),
        pltpu.VMEM_SHARED((NPAD, DQ), jnp.float32),
        pltpu.VMEM((RPS, EB), jnp.int32),
        pltpu.VMEM((EB, DQ), jnp.float32),
        pltpu.VMEM((EB, DQ), jnp.float32),
        pltpu.VMEM((EB,), jnp.int32),
        pltpu.VMEM((EB,), jnp.int32),
        pltpu.VMEM((EB,), jnp.int32),
        pltpu.VMEM((EB,), jnp.int32),
        pltpu.SemaphoreType.DMA,
        pltpu.SemaphoreType.DMA,
    ],
)
def _agg_kernel(g_hbm, packed_hbm, s_hbm, acc, gst, packed_v,
                buf0, buf1, src_u0, src_u1, dst_u0, dst_u1, sem0, sem1):
    c = lax.axis_index("c")
    s = lax.axis_index("s")
    rows = pl.ds(s * ROWS_T, ROWS_T)
    pltpu.sync_copy(packed_hbm.at[pl.ds(s * RPS, RPS)], packed_v)

    for p in range(2):
        q = 2 * c + p

        def _fill(r, carry):
            for k in range(DQ // 16):
                buf0[r, pl.ds(k * 16, 16)] = jnp.zeros((16,), jnp.float32)
            return carry

        lax.fori_loop(0, EB, _fill, 0)

        pltpu.sync_copy(g_hbm.at[rows, pl.ds(q * DQ, DQ)], gst.at[rows])
        for k in range(4):
            pltpu.sync_copy(buf0, acc.at[pl.ds(s * ROWS_T + k * EB, EB)])
        pltpu.sync_copy(buf0.at[pl.ds(0, ROWS_T - 4 * EB)],
                        acc.at[pl.ds(s * ROWS_T + 4 * EB, ROWS_T - 4 * EB)])
        plsc.subcore_barrier()

        NSPLIT = 1
        H = EB // NSPLIT

        def _descs(src_u, buf, sem):
            return [pltpu.make_async_copy(gst.at[src_u.at[pl.ds(k * H, H)]],
                                          buf.at[pl.ds(k * H, H)], sem)
                    for k in range(NSPLIT)]

        def _start(src_u, buf, sem):
            for d in _descs(src_u, buf, sem):
                d.start()

        def _wait(src_u, buf, sem):
            for d in _descs(src_u, buf, sem):
                d.wait()

        _unpack_chunk(packed_v, 0, dst_u0, src_u0, 0)
        _start(src_u0, buf0, sem0)

        def _pair(jj, carry):
            j = 2 * jj
            _unpack_chunk(packed_v, j + 1, dst_u1, src_u1, 0)
            _start(src_u1, buf1, sem1)
            _wait(src_u0, buf0, sem0)
            pltpu.sync_copy(buf0, acc.at[dst_u0], add=True)

            @pl.when(jj + 1 < RPS // 2)
            def _():
                _unpack_chunk(packed_v, j + 2, dst_u0, src_u0, 0)
                _start(src_u0, buf0, sem0)

            _wait(src_u1, buf1, sem1)
            pltpu.sync_copy(buf1, acc.at[dst_u1], add=True)
            return carry

        lax.fori_loop(0, RPS // 2, _pair, 0)
        plsc.subcore_barrier()

        pltpu.sync_copy(acc.at[rows], s_hbm.at[rows, pl.ds(q * DQ, DQ)])


def _dinv_block(deg_ref):
    d = deg_ref[0, :, 0:1] + deg_ref[1, :, 0:1] + 1.0
    return lax.rsqrt(d)


def _first_body(deg_ref, x_ref, w_ref, g_ref):
    dinv = _dinv_block(deg_ref)
    h = jnp.dot(x_ref[...], w_ref[...], preferred_element_type=jnp.float32)
    g_ref[...] = h * dinv


def _mid_body(deg_ref, s_ref, g_ref, b_ref, w_ref, o_ref):
    dinv = _dinv_block(deg_ref)
    t = s_ref[...] + g_ref[...]
    hh = jnp.maximum(t * dinv + b_ref[...][None, :], 0.0)
    h = jnp.dot(hh, w_ref[...], preferred_element_type=jnp.float32)
    o_ref[...] = h * dinv


def _last_body(deg_ref, s_ref, g_ref, b_ref, o_ref):
    dinv = _dinv_block(deg_ref)
    t = s_ref[...] + g_ref[...]
    o_ref[...] = jnp.maximum(t * dinv + b_ref[...][None, :], 0.0)


_deg_spec = pl.BlockSpec((2, BR, DEGW), lambda i: (0, i, 0))
_full_spec = pl.BlockSpec((BR, D), lambda i: (i, 0))
_w_spec = pl.BlockSpec((D, D), lambda i: (0, 0))
_b_spec = pl.BlockSpec((D,), lambda i: (0,))

_g_shape = jax.ShapeDtypeStruct((NPAD, D), jnp.float32)

_first_tc = pl.pallas_call(
    _first_body,
    grid=(NB,),
    in_specs=[_deg_spec, _full_spec, _w_spec],
    out_specs=_full_spec,
    out_shape=_g_shape,
)

_mid_tc = pl.pallas_call(
    _mid_body,
    grid=(NB,),
    in_specs=[_deg_spec, _full_spec, _full_spec, _b_spec, _w_spec],
    out_specs=_full_spec,
    out_shape=_g_shape,
)

_last_tc = pl.pallas_call(
    _last_body,
    grid=(NB,),
    in_specs=[_deg_spec, _full_spec, _full_spec, _b_spec],
    out_specs=_full_spec,
    out_shape=jax.ShapeDtypeStruct((N, D), jnp.float32),
)


def kernel(x, edge_index, W0, b0, W1, b1, W2, b2):
    npad_e = EPAD - E
    src = jnp.concatenate(
        [edge_index[0].astype(jnp.int32), jnp.zeros((npad_e,), jnp.int32)])
    dst = jnp.concatenate(
        [edge_index[1].astype(jnp.int32),
         N + (jnp.arange(npad_e, dtype=jnp.int32) % (NPAD - N))])
    packed = (src | (dst << 16)).reshape(ER, EB)

    deg2 = _deg_kernel(packed)
    g = _first_tc(deg2, x, W0)
    s = _agg_kernel(g, packed)
    g = _mid_tc(deg2, s, g, b0, W1)
    s = _agg_kernel(g, packed)
    g = _mid_tc(deg2, s, g, b1, W2)
    s = _agg_kernel(g, packed)
    return _last_tc(deg2, s, g, b2)
